# baseline (speedup 1.0000x reference)
import functools
import math

import jax
import jax.numpy as jnp
import numpy as np
from jax import lax
from jax.experimental import pallas as pl
from jax.experimental.pallas import tpu as pltpu
from jax.experimental.pallas import tpu_sc as plsc

B, NCLS, H, W = 8, 3, 512, 512
L = 200
RADIUS = 15
SIGMA = RADIUS // 3
C_CONST = float(math.log(math.e + 2.0))
INV_2SIG2 = 1.0 / (2.0 * SIGMA * SIGMA)
NROWS = RADIUS
HALF = RADIUS // 2
ROWCH = H * (W // 128)
ED = [math.exp(-((d - HALF) ** 2) * INV_2SIG2) for d in range(NROWS)]


def _colw_table():
    tab = np.zeros((16, 128), np.float32)
    for a in range(16):
        u = (a + 9) % 16
        for sl in range(32):
            q = sl - u
            if 0 <= q <= 13:
                tab[a, sl] = math.exp(-((q - 7) ** 2) * INV_2SIG2)
    return jnp.asarray(tab)


def _tc_body(ypred_ref, ytrue_ref, out_ref):
    c = pl.program_id(0) % NCLS
    x = ypred_ref[0]
    x0, x1, x2 = x[0], x[1], x[2]
    m = jnp.maximum(x0, jnp.maximum(x1, x2))
    e0 = jnp.exp(x0 - m)
    e1 = jnp.exp(x1 - m)
    e2 = jnp.exp(x2 - m)
    s = e0 + e1 + e2
    ec = jnp.where(c == 0, e0, jnp.where(c == 1, e1, e2))
    t = ytrue_ref[0, 0]
    g = jnp.where(t == c, C_CONST - ec / s, 0.0)
    out_ref[...] = g.reshape(out_ref.shape)


def _tc_field(y_pred, y_true, ib=128):
    ni = H // ib
    return pl.pallas_call(
        _tc_body,
        grid=(B * NCLS, ni),
        in_specs=[
            pl.BlockSpec((1, NCLS, ib, W), lambda p, i: (p // NCLS, 0, i, 0)),
            pl.BlockSpec((1, 1, ib, W), lambda p, i: (p // NCLS, 0, i, 0)),
        ],
        out_specs=pl.BlockSpec((ib * 4, 128), lambda p, i: (p * ni + i, 0)),
        out_shape=jax.ShapeDtypeStruct((B * NCLS * H * W // 128, 128),
                                       jnp.float32),
    )(y_pred, y_true)


def _sc_body(NC, NW, PPW, NBLK,
             g_hbm, pts_hbm, tab_hbm, out_hbm,
             pts_v, tab_v, idx_v, vals_v, acc_v, sem):
    wid = lax.axis_index("s") * NC + lax.axis_index("c")
    pltpu.sync_copy(pts_hbm.at[pl.ds(wid * (PPW // 8), PPW // 8)], pts_v)
    pltpu.sync_copy(tab_hbm, tab_v)

    lane = lax.iota(jnp.int32, 16)
    acc = jnp.zeros((16,), jnp.float32)

    for blk in range(NBLK):
        def build(p, carry):
            sp = blk * 16 + p
            pv = pts_v[sp >> 3, pl.ds((sp & 7) * 16, 16)]
            i_p = pv[0]
            j_p = pv[1]
            lab = pv[2]
            b_p = pv[3]
            ii_c = jnp.clip(i_p + lane - HALF, 0, H - 1)
            plane = b_p * NCLS + lab
            jb0 = lax.shift_right_arithmetic(j_p - HALF, 7)
            rowbase = plane * ROWCH + ii_c * 4
            for s in range(2):
                jb = jnp.clip(jb0 + s, 0, 3)
                idx_v[pl.ds(p * 32 + s * 16, 16)] = rowbase + jb
            return carry

        lax.fori_loop(0, 16, build, 0)

        copies = []
        for g in range(4):
            copies.append(pltpu.async_copy(
                g_hbm.at[idx_v.at[pl.ds(g * 128, 128)]],
                vals_v.at[pl.ds(g * 128, 128)], sem))
        for cp in copies:
            cp.wait()

        def comp(p, acc_in):
            sp = blk * 16 + p
            pv = pts_v[sp >> 3, pl.ds((sp & 7) * 16, 16)]
            i_p = pv[0]
            j_p = pv[1]
            a = j_p & 15
            cg0 = lax.shift_right_arithmetic(j_p - HALF, 4)
            jb0 = lax.shift_right_arithmetic(j_p - HALF, 7)
            coloff0 = (cg0 & 7) * 16
            cg1 = cg0 + 1
            srel1 = lax.shift_right_arithmetic(cg1, 3) - jb0
            coloff1 = (cg1 & 7) * 16
            rb0 = p * 32
            rb1 = p * 32 + srel1 * 16
            cols0 = cg0 * 16 + lane
            cols1 = cg1 * 16 + lane
            cw0 = jnp.where((cols0 >= 0) & (cols0 < W),
                            tab_v[a, pl.ds(0, 16)], 0.0)
            cw1 = jnp.where((cols1 >= 0) & (cols1 < W),
                            tab_v[a, pl.ds(16, 16)], 0.0)
            acc2 = acc_in
            for d in range(NROWS):
                ii = i_p + (d - HALF)
                rw = jnp.where((ii >= 0) & (ii < H), ED[d], 0.0)
                v0 = vals_v[rb0 + d, pl.ds(coloff0, 16)]
                v1 = vals_v[rb1 + d, pl.ds(coloff1, 16)]
                acc2 = acc2 + rw * (cw0 * v0 + cw1 * v1)
            return acc2

        acc = lax.fori_loop(0, 16, comp, acc)

    acc_v[...] = acc
    pltpu.sync_copy(acc_v, out_hbm.at[pl.ds(wid * 16, 16)])


def kernel(y_pred, y_true, points, point_labels):
    info = plsc.get_sparse_core_info()
    NC, NS = info.num_cores, info.num_subcores
    NW = NC * NS
    nblk = -(-(B * L) // (NW * 16))
    PPW = nblk * 16
    NPTS = PPW * NW

    field = _tc_field(y_pred, y_true)

    i_all = points[:, :, 0].reshape(-1)
    j_all = points[:, :, 1].reshape(-1)
    l_all = point_labels[:, :, 0].reshape(-1)
    b_all = jnp.repeat(jnp.arange(B, dtype=jnp.int32), L)
    pad = NPTS - B * L

    def prep(x, fill):
        x = jnp.concatenate([x, jnp.full((pad,), fill, jnp.int32)])
        return x.reshape(PPW, NW).T.reshape(-1)

    pts_packed = jnp.stack(
        [prep(i_all, 0), prep(j_all, -1000), prep(l_all, 0), prep(b_all, 0)]
        + [jnp.zeros((NPTS,), jnp.int32)] * 12, axis=1)
    pts_rows = pts_packed.reshape(NPTS // 8, 128)

    mesh = plsc.VectorSubcoreMesh(core_axis_name="c", subcore_axis_name="s")
    f = pl.kernel(
        functools.partial(_sc_body, NC, NW, PPW, nblk),
        out_type=jax.ShapeDtypeStruct((NW * 16,), jnp.float32),
        mesh=mesh,
        scratch_types=[
            pltpu.VMEM((PPW // 8, 128), jnp.int32),
            pltpu.VMEM((16, 128), jnp.float32),
            pltpu.VMEM((512,), jnp.int32),
            pltpu.VMEM((512, 128), jnp.float32),
            pltpu.VMEM((16,), jnp.float32),
            pltpu.SemaphoreType.DMA,
        ],
    )
    out = f(field, pts_rows, _colw_table())
    return jnp.sum(out) * (1.0 / (B * H * W))


# speedup vs baseline: 3.5596x; 3.5596x over previous
import functools
import math

import jax
import jax.numpy as jnp
import numpy as np
from jax import lax
from jax.experimental import pallas as pl
from jax.experimental.pallas import tpu as pltpu
from jax.experimental.pallas import tpu_sc as plsc

B, NCLS, H, W = 8, 3, 512, 512
L = 200
RADIUS = 15
SIGMA = RADIUS // 3
C_CONST = float(math.log(math.e + 2.0))
INV_2SIG2 = 1.0 / (2.0 * SIGMA * SIGMA)
NROWS = RADIUS
HALF = RADIUS // 2
CH16 = W // 16
ED = [math.exp(-((d - HALF) ** 2) * INV_2SIG2) for d in range(NROWS)]


def _colw_table():
    tab = np.zeros((16, 128), np.float32)
    for a in range(16):
        u = (a + 9) % 16
        for sl in range(32):
            q = sl - u
            if 0 <= q <= 13:
                tab[a, sl] = math.exp(-((q - 7) ** 2) * INV_2SIG2)
    return jnp.asarray(tab)


def _tc_body(ypred_ref, ytrue_ref, out_ref):
    x = ypred_ref[0]
    x0, x1, x2 = x[0], x[1], x[2]
    m = jnp.maximum(x0, jnp.maximum(x1, x2))
    e0 = jnp.exp(x0 - m)
    e1 = jnp.exp(x1 - m)
    e2 = jnp.exp(x2 - m)
    inv_s = 1.0 / (e0 + e1 + e2)
    t = ytrue_ref[0, 0]
    g0 = jnp.where(t == 0, C_CONST - e0 * inv_s, 0.0)
    g1 = jnp.where(t == 1, C_CONST - e1 * inv_s, 0.0)
    g2 = jnp.where(t == 2, C_CONST - e2 * inv_s, 0.0)
    stacked = jnp.stack([g0, g1, g2], axis=1)
    out_ref[...] = stacked.reshape(out_ref.shape)


def _tc_field(y_pred, y_true, ib=128):
    ni = H // ib
    return pl.pallas_call(
        _tc_body,
        grid=(B, ni),
        in_specs=[
            pl.BlockSpec((1, NCLS, ib, W), lambda b, i: (b, 0, i, 0)),
            pl.BlockSpec((1, 1, ib, W), lambda b, i: (b, 0, i, 0)),
        ],
        out_specs=pl.BlockSpec((ib * NCLS * 4, 128),
                               lambda b, i: (b * ni + i, 0)),
        out_shape=jax.ShapeDtypeStruct((B * NCLS * H * W // 128, 128),
                                       jnp.float32),
    )(y_pred, y_true)


def _sc_body(NC, NW, PPW, NBLK,
             g_hbm, pts_hbm, tab_hbm, out_hbm,
             pts_v, tab_v, idx_v, vals_v, acc_v, sem):
    wid = lax.axis_index("s") * NC + lax.axis_index("c")
    pltpu.sync_copy(pts_hbm.at[pl.ds(wid * (PPW // 8), PPW // 8)], pts_v)
    pltpu.sync_copy(tab_hbm, tab_v)

    lane = lax.iota(jnp.int32, 16)

    def build(p, carry):
        pv = pts_v[p >> 3, pl.ds((p & 7) * 16, 16)]
        i_p = pv[0]
        j_p = pv[1]
        lab = pv[2]
        b_p = pv[3]
        ii_c = jnp.clip(i_p + lane - HALF, 0, H - 1)
        cg0 = lax.shift_right_arithmetic(j_p - HALF, 4)
        rowbase = ((b_p * H + ii_c) * NCLS + lab) * CH16
        for s in range(2):
            cg = jnp.clip(cg0 + s, 0, CH16 - 1)
            idx_v[pl.ds(p * 32 + s * 16, 16)] = rowbase + cg
        return carry

    lax.fori_loop(0, PPW, build, 0)

    copies = []
    for g in range(PPW * 32 // 128):
        copies.append(pltpu.async_copy(
            g_hbm.at[idx_v.at[pl.ds(g * 128, 128)]],
            vals_v.at[pl.ds(g * 128, 128)], sem))
    for cp in copies:
        cp.wait()

    def comp(p, acc_in):
        pv = pts_v[p >> 3, pl.ds((p & 7) * 16, 16)]
        i_p = pv[0]
        j_p = pv[1]
        a = j_p & 15
        cg0 = lax.shift_right_arithmetic(j_p - HALF, 4)
        cols0 = cg0 * 16 + lane
        cols1 = cols0 + 16
        cw0 = jnp.where((cols0 >= 0) & (cols0 < W),
                        tab_v[a, pl.ds(0, 16)], 0.0)
        cw1 = jnp.where((cols1 >= 0) & (cols1 < W),
                        tab_v[a, pl.ds(16, 16)], 0.0)
        acc2 = acc_in
        for d in range(NROWS):
            ii = i_p + (d - HALF)
            rw = jnp.where((ii >= 0) & (ii < H), ED[d], 0.0)
            v0 = vals_v[p * 32 + d, :]
            v1 = vals_v[p * 32 + 16 + d, :]
            acc2 = acc2 + rw * (cw0 * v0 + cw1 * v1)
        return acc2

    acc = lax.fori_loop(0, PPW, comp, jnp.zeros((16,), jnp.float32))

    acc_v[...] = acc
    pltpu.sync_copy(acc_v, out_hbm.at[pl.ds(wid * 16, 16)])


def kernel(y_pred, y_true, points, point_labels):
    info = plsc.get_sparse_core_info()
    NC, NS = info.num_cores, info.num_subcores
    NW = NC * NS
    nblk = -(-(B * L) // (NW * 16))
    PPW = nblk * 16
    NPTS = PPW * NW

    field = _tc_field(y_pred, y_true)
    field16 = field.reshape(B * NCLS * H * W // 16, 16)

    i_all = points[:, :, 0].reshape(-1)
    j_all = points[:, :, 1].reshape(-1)
    l_all = point_labels[:, :, 0].reshape(-1)
    b_all = jnp.repeat(jnp.arange(B, dtype=jnp.int32), L)
    pad = NPTS - B * L

    def prep(x, fill):
        x = jnp.concatenate([x, jnp.full((pad,), fill, jnp.int32)])
        return x.reshape(PPW, NW).T.reshape(-1)

    pts_packed = jnp.stack(
        [prep(i_all, 0), prep(j_all, -1000), prep(l_all, 0), prep(b_all, 0)]
        + [jnp.zeros((NPTS,), jnp.int32)] * 12, axis=1)
    pts_rows = pts_packed.reshape(NPTS // 8, 128)

    mesh = plsc.VectorSubcoreMesh(core_axis_name="c", subcore_axis_name="s")
    f = pl.kernel(
        functools.partial(_sc_body, NC, NW, PPW, nblk),
        out_type=jax.ShapeDtypeStruct((NW * 16,), jnp.float32),
        mesh=mesh,
        compiler_params=pltpu.CompilerParams(use_tc_tiling_on_sc=False),
        scratch_types=[
            pltpu.VMEM((PPW // 8, 128), jnp.int32),
            pltpu.VMEM((16, 128), jnp.float32),
            pltpu.VMEM((PPW * 32,), jnp.int32),
            pltpu.VMEM((PPW * 32, 16), jnp.float32),
            pltpu.VMEM((16,), jnp.float32),
            pltpu.SemaphoreType.DMA,
        ],
    )
    out = f(field16, pts_rows, _colw_table())
    return jnp.sum(out) * (1.0 / (B * H * W))


# speedup vs baseline: 3.9045x; 1.0969x over previous
import functools
import math

import jax
import jax.numpy as jnp
import numpy as np
from jax import lax
from jax.experimental import pallas as pl
from jax.experimental.pallas import tpu as pltpu
from jax.experimental.pallas import tpu_sc as plsc

B, NCLS, H, W = 8, 3, 512, 512
L = 200
RADIUS = 15
SIGMA = RADIUS // 3
C_CONST = float(math.log(math.e + 2.0))
INV_2SIG2 = 1.0 / (2.0 * SIGMA * SIGMA)
NROWS = RADIUS
HALF = RADIUS // 2
CH16 = W // 16
ED = [math.exp(-((d - HALF) ** 2) * INV_2SIG2) for d in range(NROWS)]


def _colw_table():
    tab = np.zeros((16, 128), np.float32)
    for a in range(16):
        u = (a + 9) % 16
        for sl in range(32):
            q = sl - u
            if 0 <= q <= 13:
                tab[a, sl] = math.exp(-((q - 7) ** 2) * INV_2SIG2)
    return jnp.asarray(tab)


def _tc_body(ypred_ref, ytrue_ref, out_ref):
    x = ypred_ref[0]
    x0, x1, x2 = x[0], x[1], x[2]
    m = jnp.maximum(x0, jnp.maximum(x1, x2))
    e0 = jnp.exp(x0 - m)
    e1 = jnp.exp(x1 - m)
    e2 = jnp.exp(x2 - m)
    inv_s = 1.0 / (e0 + e1 + e2)
    t = ytrue_ref[0, 0]
    g0 = jnp.where(t == 0, C_CONST - e0 * inv_s, 0.0)
    g1 = jnp.where(t == 1, C_CONST - e1 * inv_s, 0.0)
    g2 = jnp.where(t == 2, C_CONST - e2 * inv_s, 0.0)
    stacked = jnp.stack([g0, g1, g2], axis=0)
    out_ref[...] = stacked.reshape(out_ref.shape)


def _tc_field(y_pred, y_true, ib=128):
    ni = H // ib
    return pl.pallas_call(
        _tc_body,
        grid=(B, ni),
        in_specs=[
            pl.BlockSpec((1, NCLS, ib, W), lambda b, i: (b, 0, i, 0)),
            pl.BlockSpec((1, 1, ib, W), lambda b, i: (b, 0, i, 0)),
        ],
        out_specs=pl.BlockSpec((ib * NCLS * 4, 128),
                               lambda b, i: (b * ni + i, 0)),
        out_shape=jax.ShapeDtypeStruct((B * NCLS * H * W // 128, 128),
                                       jnp.float32),
    )(y_pred, y_true)


def _sc_body(NC, NW, PPW, NBLK,
             g_hbm, pts_hbm, tab_hbm, out_hbm,
             pts_v, tab_v, idx_v, vals_v, acc_v, sem):
    wid = lax.axis_index("s") * NC + lax.axis_index("c")
    pltpu.sync_copy(pts_hbm.at[pl.ds(wid * (PPW // 8), PPW // 8)], pts_v)
    pltpu.sync_copy(tab_hbm, tab_v)

    lane = lax.iota(jnp.int32, 16)

    def build(p, carry):
        pv = pts_v[p >> 3, pl.ds((p & 7) * 16, 16)]
        i_p = pv[0]
        j_p = pv[1]
        lab = pv[2]
        b_p = pv[3]
        ii_c = jnp.clip(i_p + lane - HALF, 0, H - 1)
        cg0 = lax.shift_right_arithmetic(j_p - HALF, 4)
        iblk = lax.shift_right_arithmetic(ii_c, 7)
        li = ii_c & 127
        rowbase = ((b_p * 4 + iblk) * NCLS + lab) * 4096 + li * CH16
        for s in range(2):
            cg = jnp.clip(cg0 + s, 0, CH16 - 1)
            idx_v[pl.ds(p * 32 + s * 16, 16)] = rowbase + cg
        return carry

    lax.fori_loop(0, PPW, build, 0)

    copies = []
    for g in range(PPW * 32 // 128):
        copies.append(pltpu.async_copy(
            g_hbm.at[idx_v.at[pl.ds(g * 128, 128)]],
            vals_v.at[pl.ds(g * 128, 128)], sem))

    def comp(p, acc_in):
        pv = pts_v[p >> 3, pl.ds((p & 7) * 16, 16)]
        i_p = pv[0]
        j_p = pv[1]
        a = j_p & 15
        cg0 = lax.shift_right_arithmetic(j_p - HALF, 4)
        cols0 = cg0 * 16 + lane
        cols1 = cols0 + 16
        cw0 = jnp.where((cols0 >= 0) & (cols0 < W),
                        tab_v[a, pl.ds(0, 16)], 0.0)
        cw1 = jnp.where((cols1 >= 0) & (cols1 < W),
                        tab_v[a, pl.ds(16, 16)], 0.0)
        acc2 = acc_in
        for d in range(NROWS):
            ii = i_p + (d - HALF)
            rw = jnp.where((ii >= 0) & (ii < H), ED[d], 0.0)
            v0 = vals_v[p * 32 + d, :]
            v1 = vals_v[p * 32 + 16 + d, :]
            acc2 = acc2 + rw * (cw0 * v0 + cw1 * v1)
        return acc2

    acc = jnp.zeros((16,), jnp.float32)
    for g in range(PPW * 32 // 128):
        copies[g].wait()
        acc = lax.fori_loop(g * 4, g * 4 + 4, comp, acc)

    acc_v[...] = acc
    pltpu.sync_copy(acc_v, out_hbm.at[pl.ds(wid * 16, 16)])


def kernel(y_pred, y_true, points, point_labels):
    info = plsc.get_sparse_core_info()
    NC, NS = info.num_cores, info.num_subcores
    NW = NC * NS
    nblk = -(-(B * L) // (NW * 16))
    PPW = nblk * 16
    NPTS = PPW * NW

    field = _tc_field(y_pred, y_true)
    field16 = field.reshape(B * NCLS * H * W // 16, 16)

    i_all = points[:, :, 0].reshape(-1)
    j_all = points[:, :, 1].reshape(-1)
    l_all = point_labels[:, :, 0].reshape(-1)
    b_all = jnp.repeat(jnp.arange(B, dtype=jnp.int32), L)
    pad = NPTS - B * L

    def prep(x, fill):
        x = jnp.concatenate([x, jnp.full((pad,), fill, jnp.int32)])
        return x.reshape(PPW, NW).T.reshape(-1)

    pts_packed = jnp.stack(
        [prep(i_all, 0), prep(j_all, -1000), prep(l_all, 0), prep(b_all, 0)]
        + [jnp.zeros((NPTS,), jnp.int32)] * 12, axis=1)
    pts_rows = pts_packed.reshape(NPTS // 8, 128)

    mesh = plsc.VectorSubcoreMesh(core_axis_name="c", subcore_axis_name="s")
    f = pl.kernel(
        functools.partial(_sc_body, NC, NW, PPW, nblk),
        out_type=jax.ShapeDtypeStruct((NW * 16,), jnp.float32),
        mesh=mesh,
        compiler_params=pltpu.CompilerParams(use_tc_tiling_on_sc=False),
        scratch_types=[
            pltpu.VMEM((PPW // 8, 128), jnp.int32),
            pltpu.VMEM((16, 128), jnp.float32),
            pltpu.VMEM((PPW * 32,), jnp.int32),
            pltpu.VMEM((PPW * 32, 16), jnp.float32),
            pltpu.VMEM((16,), jnp.float32),
            pltpu.SemaphoreType.DMA,
        ],
    )
    out = f(field16, pts_rows, _colw_table())
    return jnp.sum(out) * (1.0 / (B * H * W))


# speedup vs baseline: 4.0650x; 1.0411x over previous
import functools
import math

import jax
import jax.numpy as jnp
import numpy as np
from jax import lax
from jax.experimental import pallas as pl
from jax.experimental.pallas import tpu as pltpu
from jax.experimental.pallas import tpu_sc as plsc

B, NCLS, H, W = 8, 3, 512, 512
L = 200
RADIUS = 15
SIGMA = RADIUS // 3
C_CONST = float(math.log(math.e + 2.0))
INV_2SIG2 = 1.0 / (2.0 * SIGMA * SIGMA)
NROWS = RADIUS
HALF = RADIUS // 2
CH16 = W // 16
ED = [math.exp(-((d - HALF) ** 2) * INV_2SIG2) for d in range(NROWS)]


def _colw_table():
    tab = np.zeros((16, 128), np.float32)
    for a in range(16):
        u = (a + 9) % 16
        for sl in range(32):
            q = sl - u
            if 0 <= q <= 13:
                tab[a, sl] = math.exp(-((q - 7) ** 2) * INV_2SIG2)
    return jnp.asarray(tab)


def _tc_body(ypred_ref, ytrue_ref, out_ref):
    x = ypred_ref[0]
    x0, x1, x2 = x[0], x[1], x[2]
    m = jnp.maximum(x0, jnp.maximum(x1, x2))
    e0 = jnp.exp(x0 - m)
    e1 = jnp.exp(x1 - m)
    e2 = jnp.exp(x2 - m)
    inv_s = 1.0 / (e0 + e1 + e2)
    t = ytrue_ref[0, 0]
    g0 = jnp.where(t == 0, C_CONST - e0 * inv_s, 0.0)
    g1 = jnp.where(t == 1, C_CONST - e1 * inv_s, 0.0)
    g2 = jnp.where(t == 2, C_CONST - e2 * inv_s, 0.0)
    stacked = jnp.stack([g0, g1, g2], axis=0)
    out_ref[...] = stacked.reshape(out_ref.shape)


def _tc_field(y_pred, y_true, ib=256):
    ni = H // ib
    return pl.pallas_call(
        _tc_body,
        grid=(B, ni),
        in_specs=[
            pl.BlockSpec((1, NCLS, ib, W), lambda b, i: (b, 0, i, 0)),
            pl.BlockSpec((1, 1, ib, W), lambda b, i: (b, 0, i, 0)),
        ],
        out_specs=pl.BlockSpec((ib * NCLS * 4, 128),
                               lambda b, i: (b * ni + i, 0)),
        out_shape=jax.ShapeDtypeStruct((B * NCLS * H * W // 128, 128),
                                       jnp.float32),
    )(y_pred, y_true)


def _sc_body(NC, NW, PPW, NBLK,
             g_hbm, pts_hbm, tab_hbm, out_hbm,
             pts_v, tab_v, idx_v, vals_v, acc_v, sem):
    wid = lax.axis_index("s") * NC + lax.axis_index("c")
    pltpu.sync_copy(pts_hbm.at[pl.ds(wid * (PPW // 8), PPW // 8)], pts_v)
    pltpu.sync_copy(tab_hbm, tab_v)

    lane = lax.iota(jnp.int32, 16)

    def build(p, carry):
        pv = pts_v[p >> 3, pl.ds((p & 7) * 16, 16)]
        i_p = pv[0]
        j_p = pv[1]
        lab = pv[2]
        b_p = pv[3]
        ii_c = jnp.clip(i_p + lane - HALF, 0, H - 1)
        cg0 = lax.shift_right_arithmetic(j_p - HALF, 4)
        iblk = lax.shift_right_arithmetic(ii_c, 7)
        li = ii_c & 127
        rowbase = ((b_p * 4 + iblk) * NCLS + lab) * 4096 + li * CH16
        for s in range(2):
            cg = jnp.clip(cg0 + s, 0, CH16 - 1)
            idx_v[pl.ds(p * 32 + s * 16, 16)] = rowbase + cg
        return carry

    lax.fori_loop(0, PPW, build, 0)

    copies = []
    for g in range(PPW * 32 // 128):
        copies.append(pltpu.async_copy(
            g_hbm.at[idx_v.at[pl.ds(g * 128, 128)]],
            vals_v.at[pl.ds(g * 128, 128)], sem))

    def comp(p, acc_in):
        pv = pts_v[p >> 3, pl.ds((p & 7) * 16, 16)]
        i_p = pv[0]
        j_p = pv[1]
        a = j_p & 15
        cg0 = lax.shift_right_arithmetic(j_p - HALF, 4)
        cols0 = cg0 * 16 + lane
        cols1 = cols0 + 16
        cw0 = jnp.where((cols0 >= 0) & (cols0 < W),
                        tab_v[a, pl.ds(0, 16)], 0.0)
        cw1 = jnp.where((cols1 >= 0) & (cols1 < W),
                        tab_v[a, pl.ds(16, 16)], 0.0)
        acc2 = acc_in
        for d in range(NROWS):
            ii = i_p + (d - HALF)
            rw = jnp.where((ii >= 0) & (ii < H), ED[d], 0.0)
            v0 = vals_v[p * 32 + d, :]
            v1 = vals_v[p * 32 + 16 + d, :]
            acc2 = acc2 + rw * (cw0 * v0 + cw1 * v1)
        return acc2

    acc = jnp.zeros((16,), jnp.float32)
    for g in range(PPW * 32 // 128):
        copies[g].wait()
        acc = lax.fori_loop(g * 4, g * 4 + 4, comp, acc)

    acc_v[...] = acc
    pltpu.sync_copy(acc_v, out_hbm.at[pl.ds(wid * 16, 16)])


def kernel(y_pred, y_true, points, point_labels):
    info = plsc.get_sparse_core_info()
    NC, NS = info.num_cores, info.num_subcores
    NW = NC * NS
    nblk = -(-(B * L) // (NW * 16))
    PPW = nblk * 16
    NPTS = PPW * NW

    field = _tc_field(y_pred, y_true.astype(jnp.int8))
    field16 = field.reshape(B * NCLS * H * W // 16, 16)

    i_all = points[:, :, 0].reshape(-1)
    j_all = points[:, :, 1].reshape(-1)
    l_all = point_labels[:, :, 0].reshape(-1)
    b_all = jnp.repeat(jnp.arange(B, dtype=jnp.int32), L)
    pad = NPTS - B * L

    def prep(x, fill):
        x = jnp.concatenate([x, jnp.full((pad,), fill, jnp.int32)])
        return x.reshape(PPW, NW).T.reshape(-1)

    pts_packed = jnp.stack(
        [prep(i_all, 0), prep(j_all, -1000), prep(l_all, 0), prep(b_all, 0)]
        + [jnp.zeros((NPTS,), jnp.int32)] * 12, axis=1)
    pts_rows = pts_packed.reshape(NPTS // 8, 128)

    mesh = plsc.VectorSubcoreMesh(core_axis_name="c", subcore_axis_name="s")
    f = pl.kernel(
        functools.partial(_sc_body, NC, NW, PPW, nblk),
        out_type=jax.ShapeDtypeStruct((NW * 16,), jnp.float32),
        mesh=mesh,
        compiler_params=pltpu.CompilerParams(use_tc_tiling_on_sc=False),
        scratch_types=[
            pltpu.VMEM((PPW // 8, 128), jnp.int32),
            pltpu.VMEM((16, 128), jnp.float32),
            pltpu.VMEM((PPW * 32,), jnp.int32),
            pltpu.VMEM((PPW * 32, 16), jnp.float32),
            pltpu.VMEM((16,), jnp.float32),
            pltpu.SemaphoreType.DMA,
        ],
    )
    out = f(field16, pts_rows, _colw_table())
    return jnp.sum(out) * (1.0 / (B * H * W))
